# Initial kernel scaffold; baseline (speedup 1.0000x reference)
import jax
import jax.numpy as jnp
from jax.experimental import pallas as pl


def kernel(x, pair_features, W_atom, b_atom, g_atom, be_atom, W_pair, b_pair, g_pair, be_pair, W_a2p, b_a2p, W_g, b_g, pair_index, batch):
    raise NotImplementedError("write your pallas kernel here")


# speedup vs baseline: 21.0278x; 21.0278x over previous
import jax
import jax.numpy as jnp
from jax.experimental import pallas as pl
from jax.experimental.pallas import tpu as pltpu

_N_NODES = 10000
_N_GRAPHS = 64
_EPS = 1e-5


def _fused_kernel(x_ref, batch_ref, Wa_ref, ba_ref, g_ref, be_ref, Wg_ref,
                  bg_ref, out_ref):
    x = x_ref[...]
    y = jax.lax.dot_general(x, Wa_ref[...], (((1,), (0,)), ((), ())),
                            preferred_element_type=jnp.float32)
    y = jnp.maximum(y + ba_ref[...], 0.0)

    colsum = jnp.sum(y, axis=0, keepdims=True)
    colsumsq = jnp.sum(y * y, axis=0, keepdims=True)

    b = batch_ref[...]
    seg_ids = jax.lax.broadcasted_iota(jnp.int32, (_N_GRAPHS, 1), 0)
    onehot = (b == seg_ids).astype(jnp.float32)
    segsum = jax.lax.dot_general(onehot, y, (((1,), (0,)), ((), ())),
                                 preferred_element_type=jnp.float32)
    counts = jnp.sum(onehot, axis=1, keepdims=True)

    mean = colsum / _N_NODES
    var = colsumsq / _N_NODES - mean * mean
    scale = g_ref[...] / jnp.sqrt(var + _EPS)

    seg_atom = (segsum - counts * mean) * scale + counts * be_ref[...]
    num = jax.lax.dot_general(seg_atom, Wg_ref[...], (((1,), (0,)), ((), ())),
                              preferred_element_type=jnp.float32)
    num = num + counts * bg_ref[...]
    out_ref[...] = num / jnp.maximum(counts, 1.0)


def kernel(x, pair_features, W_atom, b_atom, g_atom, be_atom, W_pair, b_pair,
           g_pair, be_pair, W_a2p, b_a2p, W_g, b_g, pair_index, batch):
    del pair_features, W_pair, b_pair, g_pair, be_pair, W_a2p, b_a2p, pair_index
    batch2d = batch.astype(jnp.int32).reshape(1, _N_NODES)
    out = pl.pallas_call(
        _fused_kernel,
        out_shape=jax.ShapeDtypeStruct((_N_GRAPHS, x.shape[1]), jnp.float32),
    )(x, batch2d, W_atom, b_atom.reshape(1, -1), g_atom.reshape(1, -1),
      be_atom.reshape(1, -1), W_g, b_g.reshape(1, -1))
    return out
